# Initial kernel scaffold; baseline (speedup 1.0000x reference)
import jax
import jax.numpy as jnp
from jax.experimental import pallas as pl


def kernel(x, edge_index, W1, b1, gamma, beta, W2, b2):
    raise NotImplementedError("write your pallas kernel here")


# speedup vs baseline: 3.7892x; 3.7892x over previous
import functools

import jax
import jax.numpy as jnp
from jax import lax
from jax.experimental import pallas as pl
from jax.experimental.pallas import tpu as pltpu
from jax.experimental.pallas import tpu_sc as plsc

N = 10000
D = 128
H = 128
E = 320000

NC = 2
NS = 16
NW = NC * NS
DH = D // 2
NP = 10240
RNG = NP // NS
P = 80
EPW = E // NW
BPW = EPW // P
EB = 2000
NEB = E // EB
C = 1024
GB = 128

_mesh = plsc.VectorSubcoreMesh(core_axis_name="c", subcore_axis_name="s")


@functools.partial(
    pl.kernel,
    out_type=jax.ShapeDtypeStruct((NW, NP // 128, 128), jnp.float32),
    mesh=_mesh,
    compiler_params=pltpu.CompilerParams(needs_layout_passes=False),
    scratch_types=[
        pltpu.VMEM((BPW, P), jnp.int32),
        pltpu.VMEM((NP // 128, 128), jnp.float32),
    ],
)
def _sc_degree(dst2_hbm, deg_hbm, dst_v, hist_v):
    c = lax.axis_index("c")
    s = lax.axis_index("s")
    w = c * NS + s

    zero16 = jnp.zeros((16,), dtype=jnp.float32)
    one16 = jnp.full((16,), 1.0, dtype=jnp.float32)
    i16 = jnp.arange(16, dtype=jnp.int32)

    @pl.loop(0, NP // 128)
    def _(i):
        for k in range(8):
            hist_v[i, pl.ds(k * 16, 16)] = zero16

    pltpu.sync_copy(dst2_hbm.at[w], dst_v)

    @pl.loop(0, BPW)
    def _(r):
        for k in range(P // 16):
            d16 = dst_v[r, pl.ds(k * 16, 16)]
            row = lax.shift_right_logical(d16, 7)
            col = lax.bitwise_and(d16, 127)
            for lane in range(16):
                plsc.addupdate_scatter(
                    hist_v, [row, col], one16, mask=(i16 == lane)
                )

    pltpu.sync_copy(hist_v, deg_hbm.at[w])


@functools.partial(
    pl.kernel,
    out_type=jax.ShapeDtypeStruct((NC, NP, DH), jnp.float32),
    mesh=_mesh,
    compiler_params=pltpu.CompilerParams(
        use_tc_tiling_on_sc=False, needs_layout_passes=False
    ),
    scratch_types=[
        pltpu.VMEM((EB,), jnp.int32),
        pltpu.VMEM((EB,), jnp.int32),
        pltpu.VMEM((C,), jnp.int32),
        pltpu.VMEM((C,), jnp.int32),
        pltpu.VMEM((GB, DH), jnp.float32),
        pltpu.VMEM((RNG, DH), jnp.float32),
        pltpu.SemaphoreType.DMA,
    ],
)
def _sc_propagate(tab_hbm, src_hbm, dst_hbm, out_hbm,
                  se_v, de_v, cs_v, cl_v, rows_v, acc_v, sem):
    c = lax.axis_index("c")
    s = lax.axis_index("s")
    lo = s * RNG

    zero16 = jnp.zeros((16,), dtype=jnp.float32)
    zero16i = jnp.zeros((16,), dtype=jnp.int32)

    @pl.loop(0, RNG)
    def _(i):
        for k in range(DH // 16):
            acc_v[i, pl.ds(k * 16, 16)] = zero16

    @pl.loop(0, C // 16)
    def _(i):
        cs_v[pl.ds(i * 16, 16)] = zero16i

    hc = tab_hbm.at[c]

    i16 = jnp.arange(16, dtype=jnp.int32)

    def flush(cnt):
        nb = lax.div(cnt + (GB - 1), GB)

        @pl.loop(0, nb)
        def _(b):
            base = b * GB
            pltpu.async_copy(hc.at[cs_v.at[pl.ds(base, GB)]], rows_v, sem).wait()
            nin = jnp.minimum(cnt - base, GB)

            @pl.loop(0, nin, step=16)
            def _(i0):
                ld16 = cl_v[pl.ds(base + i0, 16)]
                nlane = jnp.minimum(nin - i0, 16)

                @pl.loop(0, nlane)
                def _(lane):
                    ld = jnp.sum(jnp.where(i16 == lane, ld16, 0))
                    for f in range(DH // 16):
                        plsc.addupdate(
                            acc_v.at[ld, pl.ds(16 * f, 16)],
                            rows_v[i0 + lane, pl.ds(16 * f, 16)],
                        )

    @pl.loop(0, NEB, init_carry=jnp.int32(0))
    def scan(b, cnt):
        pltpu.sync_copy(src_hbm.at[pl.ds(b * EB, EB)], se_v)
        pltpu.sync_copy(dst_hbm.at[pl.ds(b * EB, EB)], de_v)

        @pl.loop(0, EB // 16, init_carry=cnt)
        def vstep(v, cnt):
            full = cnt >= C - 16

            @pl.when(full)
            def _():
                flush(cnt)

            cnt = jnp.where(full, 0, cnt)
            s16 = se_v[pl.ds(v * 16, 16)]
            d16 = de_v[pl.ds(v * 16, 16)]
            ld16 = d16 - lo
            m = (ld16 >= 0) & (ld16 < RNG)
            plsc.store_compressed(cs_v.at[pl.ds(cnt, 16)], s16, mask=m)
            plsc.store_compressed(cl_v.at[pl.ds(cnt, 16)], ld16, mask=m)
            return cnt + jnp.sum(m.astype(jnp.int32))

        return vstep

    @pl.when(scan > 0)
    def _():
        flush(scan)

    pltpu.sync_copy(acc_v, out_hbm.at[c, pl.ds(s * RNG, RNG)])


_RB = 1024
_GRID = (N + _RB - 1) // _RB


def _dinv_from_deg(degp_blk):
    deg = jnp.sum(degp_blk, axis=0) + 1.0
    return lax.rsqrt(deg)


def _halves(ref):
    return jnp.concatenate([ref[0], ref[1]], axis=-1)


def _store_halves(o_ref, v):
    o_ref[0] = v[:, :DH]
    o_ref[1] = v[:, DH:]


def _tc_mm_scale_body(x_ref, w_ref, degp_ref, o_ref):
    dinv = _dinv_from_deg(degp_ref[...])
    h = jnp.dot(x_ref[...], w_ref[...], preferred_element_type=jnp.float32)
    _store_halves(o_ref, dinv[:, None] * h)


def _tc_mid_body(acc_ref, hp_ref, degp_ref, b1_ref, g_ref, be_ref, w2_ref,
                 o_ref):
    dinv = _dinv_from_deg(degp_ref[...])
    acc = _halves(acc_ref) + _halves(hp_ref)
    z = dinv[:, None] * acc + b1_ref[...]
    mu = jnp.mean(z, axis=-1, keepdims=True)
    zc = z - mu
    var = jnp.mean(zc * zc, axis=-1, keepdims=True)
    ln = zc * lax.rsqrt(var + 1e-5) * g_ref[...] + be_ref[...]
    g = jnp.maximum(ln, 0.0)
    h2 = jnp.dot(g, w2_ref[...], preferred_element_type=jnp.float32)
    _store_halves(o_ref, dinv[:, None] * h2)


def _tc_final_body(acc_ref, hp_ref, degp_ref, b2_ref, o_ref):
    dinv = _dinv_from_deg(degp_ref[...])
    acc = _halves(acc_ref) + _halves(hp_ref)
    o_ref[...] = jnp.maximum(dinv[:, None] * acc + b2_ref[...], 0.0)


_rowspec = pl.BlockSpec((_RB, D), lambda i: (i, 0))
_halfspec = pl.BlockSpec((NC, _RB, DH), lambda i: (0, i, 0))
_degspec = pl.BlockSpec((NW, _RB), lambda i: (0, i))
_wspec = pl.BlockSpec((D, H), lambda i: (0, 0))
_vspec = pl.BlockSpec((1, D), lambda i: (0, 0))
_out_half = jax.ShapeDtypeStruct((NC, N, DH), jnp.float32)

_tc_mm_scale = pl.pallas_call(
    _tc_mm_scale_body,
    grid=(_GRID,),
    in_specs=[_rowspec, _wspec, _degspec],
    out_specs=_halfspec,
    out_shape=_out_half,
)

_tc_mid = pl.pallas_call(
    _tc_mid_body,
    grid=(_GRID,),
    in_specs=[_halfspec, _halfspec, _degspec, _vspec, _vspec, _vspec, _wspec],
    out_specs=_halfspec,
    out_shape=_out_half,
)

_tc_final = pl.pallas_call(
    _tc_final_body,
    grid=(_GRID,),
    in_specs=[_halfspec, _halfspec, _degspec, _vspec],
    out_specs=_rowspec,
    out_shape=jax.ShapeDtypeStruct((N, D), jnp.float32),
)


def kernel(x, edge_index, W1, b1, gamma, beta, W2, b2):
    srcf = edge_index[0]
    dstf = edge_index[1]
    dst2 = dstf.reshape(NW, BPW, P)
    b1r = b1.reshape(1, H)
    gr = gamma.reshape(1, H)
    ber = beta.reshape(1, H)
    b2r = b2.reshape(1, D)

    degp = _sc_degree(dst2).reshape(NW, NP)
    h1p = _tc_mm_scale(x, W1, degp)
    acc1 = _sc_propagate(h1p, srcf, dstf)
    h2p = _tc_mid(acc1, h1p, degp, b1r, gr, ber, W2)
    acc2 = _sc_propagate(h2p, srcf, dstf)
    return _tc_final(acc2, h2p, degp, b2r)


# speedup vs baseline: 4.5604x; 1.2035x over previous
import functools

import jax
import jax.numpy as jnp
from jax import lax
from jax.experimental import pallas as pl
from jax.experimental.pallas import tpu as pltpu
from jax.experimental.pallas import tpu_sc as plsc

N = 10000
D = 128
H = 128
E = 320000

NC = 2
NS = 16
NW = NC * NS
DH = D // 2
NP = 10240
RNG = NP // NS
P = 80
EPW = E // NW
BPW = EPW // P
EB = 2000
NEB = E // EB
C = 1024
GB = 128

_mesh = plsc.VectorSubcoreMesh(core_axis_name="c", subcore_axis_name="s")


@functools.partial(
    pl.kernel,
    out_type=jax.ShapeDtypeStruct((NW, NP // 128, 128), jnp.float32),
    mesh=_mesh,
    compiler_params=pltpu.CompilerParams(needs_layout_passes=False),
    scratch_types=[
        pltpu.VMEM((BPW, P), jnp.int32),
        pltpu.VMEM((NP // 128, 128), jnp.float32),
    ],
)
def _sc_degree(dst2_hbm, deg_hbm, dst_v, hist_v):
    c = lax.axis_index("c")
    s = lax.axis_index("s")
    w = c * NS + s

    zero16 = jnp.zeros((16,), dtype=jnp.float32)
    one16 = jnp.full((16,), 1.0, dtype=jnp.float32)
    i16 = jnp.arange(16, dtype=jnp.int32)

    @pl.loop(0, NP // 128)
    def _(i):
        for k in range(8):
            hist_v[i, pl.ds(k * 16, 16)] = zero16

    pltpu.sync_copy(dst2_hbm.at[w], dst_v)

    @pl.loop(0, BPW)
    def _(r):
        for k in range(P // 16):
            d16 = dst_v[r, pl.ds(k * 16, 16)]
            row = lax.shift_right_logical(d16, 7)
            col = lax.bitwise_and(d16, 127)
            for lane in range(16):
                plsc.addupdate_scatter(
                    hist_v, [row, col], one16, mask=(i16 == lane)
                )

    pltpu.sync_copy(hist_v, deg_hbm.at[w])


@functools.partial(
    pl.kernel,
    out_type=jax.ShapeDtypeStruct((NC, NP, DH), jnp.float32),
    mesh=_mesh,
    compiler_params=pltpu.CompilerParams(
        use_tc_tiling_on_sc=False, needs_layout_passes=False
    ),
    scratch_types=[
        [pltpu.VMEM((EB,), jnp.int32) for _ in range(2)],
        [pltpu.VMEM((EB,), jnp.int32) for _ in range(2)],
        pltpu.VMEM((C,), jnp.int32),
        pltpu.VMEM((C,), jnp.int32),
        pltpu.VMEM((GB, DH), jnp.float32),
        pltpu.VMEM((RNG, DH), jnp.float32),
        pltpu.SemaphoreType.DMA,
        [pltpu.SemaphoreType.DMA for _ in range(2)],
    ],
)
def _sc_propagate(tab_hbm, src_hbm, dst_hbm, out_hbm,
                  se_b, de_b, cs_v, cl_v, rows_v, acc_v, sem, ssem):
    c = lax.axis_index("c")
    s = lax.axis_index("s")
    lo = s * RNG

    zero16 = jnp.zeros((16,), dtype=jnp.float32)
    zero16i = jnp.zeros((16,), dtype=jnp.int32)

    @pl.loop(0, RNG)
    def _(i):
        for k in range(DH // 16):
            acc_v[i, pl.ds(k * 16, 16)] = zero16

    @pl.loop(0, C // 16)
    def _(i):
        cs_v[pl.ds(i * 16, 16)] = zero16i

    hc = tab_hbm.at[c]

    i16 = jnp.arange(16, dtype=jnp.int32)

    def flush(cnt):
        nb = lax.div(cnt + (GB - 1), GB)

        @pl.loop(0, nb)
        def _(b):
            base = b * GB
            pltpu.async_copy(hc.at[cs_v.at[pl.ds(base, GB)]], rows_v, sem).wait()
            nin = jnp.minimum(cnt - base, GB)

            @pl.loop(0, nin, step=16)
            def _(i0):
                ld16 = cl_v[pl.ds(base + i0, 16)]
                nlane = nin - i0
                for lane in range(16):
                    @pl.when(lane < nlane)
                    def _(lane=lane):
                        ld = ld16[lane]
                        for f in range(DH // 16):
                            plsc.addupdate(
                                acc_v.at[ld, pl.ds(16 * f, 16)],
                                rows_v[i0 + lane, pl.ds(16 * f, 16)],
                            )

    pltpu.async_copy(src_hbm.at[pl.ds(0, EB)], se_b[0], ssem[0])
    pltpu.async_copy(dst_hbm.at[pl.ds(0, EB)], de_b[0], ssem[0])

    @pl.loop(0, NEB // 2, init_carry=jnp.int32(0))
    def scan(g, cnt):
        for j in range(2):
            b = g * 2 + j
            jn = 1 - j
            pltpu.make_async_copy(src_hbm.at[pl.ds(0, EB)], se_b[j],
                                  ssem[j]).wait()
            pltpu.make_async_copy(dst_hbm.at[pl.ds(0, EB)], de_b[j],
                                  ssem[j]).wait()

            @pl.when(b + 1 < NEB)
            def _():
                pltpu.async_copy(src_hbm.at[pl.ds((b + 1) * EB, EB)],
                                 se_b[jn], ssem[jn])
                pltpu.async_copy(dst_hbm.at[pl.ds((b + 1) * EB, EB)],
                                 de_b[jn], ssem[jn])

            se_v, de_v = se_b[j], de_b[j]

            @pl.loop(0, EB // 16, init_carry=cnt)
            def vstep(v, cnt):
                full = cnt >= C - 16

                @pl.when(full)
                def _():
                    flush(cnt)

                cnt = jnp.where(full, 0, cnt)
                s16 = se_v[pl.ds(v * 16, 16)]
                d16 = de_v[pl.ds(v * 16, 16)]
                ld16 = d16 - lo
                m = (ld16 >= 0) & (ld16 < RNG)
                plsc.store_compressed(cs_v.at[pl.ds(cnt, 16)], s16, mask=m)
                plsc.store_compressed(cl_v.at[pl.ds(cnt, 16)], ld16, mask=m)
                return cnt + jnp.sum(m.astype(jnp.int32))

            cnt = vstep
        return cnt

    @pl.when(scan > 0)
    def _():
        flush(scan)

    pltpu.sync_copy(acc_v, out_hbm.at[c, pl.ds(s * RNG, RNG)])


_RB = 1024
_GRID = (N + _RB - 1) // _RB


def _dinv_from_deg(degp_blk):
    deg = jnp.sum(degp_blk, axis=0) + 1.0
    return lax.rsqrt(deg)


def _halves(ref):
    return jnp.concatenate([ref[0], ref[1]], axis=-1)


def _store_halves(o_ref, v):
    o_ref[0] = v[:, :DH]
    o_ref[1] = v[:, DH:]


def _tc_mm_scale_body(x_ref, w_ref, degp_ref, o_ref):
    dinv = _dinv_from_deg(degp_ref[...])
    h = jnp.dot(x_ref[...], w_ref[...], preferred_element_type=jnp.float32)
    _store_halves(o_ref, dinv[:, None] * h)


def _tc_mid_body(acc_ref, hp_ref, degp_ref, b1_ref, g_ref, be_ref, w2_ref,
                 o_ref):
    dinv = _dinv_from_deg(degp_ref[...])
    acc = _halves(acc_ref) + _halves(hp_ref)
    z = dinv[:, None] * acc + b1_ref[...]
    mu = jnp.mean(z, axis=-1, keepdims=True)
    zc = z - mu
    var = jnp.mean(zc * zc, axis=-1, keepdims=True)
    ln = zc * lax.rsqrt(var + 1e-5) * g_ref[...] + be_ref[...]
    g = jnp.maximum(ln, 0.0)
    h2 = jnp.dot(g, w2_ref[...], preferred_element_type=jnp.float32)
    _store_halves(o_ref, dinv[:, None] * h2)


def _tc_final_body(acc_ref, hp_ref, degp_ref, b2_ref, o_ref):
    dinv = _dinv_from_deg(degp_ref[...])
    acc = _halves(acc_ref) + _halves(hp_ref)
    o_ref[...] = jnp.maximum(dinv[:, None] * acc + b2_ref[...], 0.0)


_rowspec = pl.BlockSpec((_RB, D), lambda i: (i, 0))
_halfspec = pl.BlockSpec((NC, _RB, DH), lambda i: (0, i, 0))
_degspec = pl.BlockSpec((NW, _RB), lambda i: (0, i))
_wspec = pl.BlockSpec((D, H), lambda i: (0, 0))
_vspec = pl.BlockSpec((1, D), lambda i: (0, 0))
_out_half = jax.ShapeDtypeStruct((NC, N, DH), jnp.float32)

_tc_mm_scale = pl.pallas_call(
    _tc_mm_scale_body,
    grid=(_GRID,),
    in_specs=[_rowspec, _wspec, _degspec],
    out_specs=_halfspec,
    out_shape=_out_half,
)

_tc_mid = pl.pallas_call(
    _tc_mid_body,
    grid=(_GRID,),
    in_specs=[_halfspec, _halfspec, _degspec, _vspec, _vspec, _vspec, _wspec],
    out_specs=_halfspec,
    out_shape=_out_half,
)

_tc_final = pl.pallas_call(
    _tc_final_body,
    grid=(_GRID,),
    in_specs=[_halfspec, _halfspec, _degspec, _vspec],
    out_specs=_rowspec,
    out_shape=jax.ShapeDtypeStruct((N, D), jnp.float32),
)


def kernel(x, edge_index, W1, b1, gamma, beta, W2, b2):
    srcf = edge_index[0]
    dstf = edge_index[1]
    dst2 = dstf.reshape(NW, BPW, P)
    b1r = b1.reshape(1, H)
    gr = gamma.reshape(1, H)
    ber = beta.reshape(1, H)
    b2r = b2.reshape(1, D)

    degp = _sc_degree(dst2).reshape(NW, NP)
    h1p = _tc_mm_scale(x, W1, degp)
    acc1 = _sc_propagate(h1p, srcf, dstf)
    h2p = _tc_mid(acc1, h1p, degp, b1r, gr, ber, W2)
    acc2 = _sc_propagate(h2p, srcf, dstf)
    return _tc_final(acc2, h2p, degp, b2r)


# speedup vs baseline: 5.5403x; 1.2149x over previous
import functools

import jax
import jax.numpy as jnp
from jax import lax
from jax.experimental import pallas as pl
from jax.experimental.pallas import tpu as pltpu
from jax.experimental.pallas import tpu_sc as plsc

N = 10000
D = 128
H = 128
E = 320000

NC = 2
NS = 16
NW = NC * NS
DH = D // 2
NP = 10240
RNG = NP // NS
P = 80
EPW = E // NW
BPW = EPW // P
EB = 2000
NEB = E // EB
C = 1024
GB = 128

_mesh = plsc.VectorSubcoreMesh(core_axis_name="c", subcore_axis_name="s")


@functools.partial(
    pl.kernel,
    out_type=jax.ShapeDtypeStruct((NW, NP // 128, 128), jnp.float32),
    mesh=_mesh,
    compiler_params=pltpu.CompilerParams(needs_layout_passes=False),
    scratch_types=[
        pltpu.VMEM((BPW, P), jnp.int32),
        pltpu.VMEM((NP // 128, 128), jnp.float32),
    ],
)
def _sc_degree(dst2_hbm, deg_hbm, dst_v, hist_v):
    c = lax.axis_index("c")
    s = lax.axis_index("s")
    w = c * NS + s

    zero16 = jnp.zeros((16,), dtype=jnp.float32)
    one16 = jnp.full((16,), 1.0, dtype=jnp.float32)
    i16 = jnp.arange(16, dtype=jnp.int32)

    @pl.loop(0, NP // 128)
    def _(i):
        for k in range(8):
            hist_v[i, pl.ds(k * 16, 16)] = zero16

    pltpu.sync_copy(dst2_hbm.at[w], dst_v)

    @pl.loop(0, BPW)
    def _(r):
        for k in range(P // 16):
            d16 = dst_v[r, pl.ds(k * 16, 16)]
            row = lax.shift_right_logical(d16, 7)
            col = lax.bitwise_and(d16, 127)
            for lane in range(16):
                plsc.addupdate_scatter(
                    hist_v, [row, col], one16, mask=(i16 == lane)
                )

    pltpu.sync_copy(hist_v, deg_hbm.at[w])


@functools.partial(
    pl.kernel,
    out_type=jax.ShapeDtypeStruct((NC, NP, DH), jnp.float32),
    mesh=_mesh,
    compiler_params=pltpu.CompilerParams(
        use_tc_tiling_on_sc=False, needs_layout_passes=False
    ),
    scratch_types=[
        [pltpu.VMEM((EB,), jnp.int32) for _ in range(2)],
        [pltpu.VMEM((EB,), jnp.int32) for _ in range(2)],
        pltpu.VMEM((C,), jnp.int32),
        pltpu.VMEM((C,), jnp.int32),
        [pltpu.VMEM((GB, DH), jnp.float32) for _ in range(2)],
        pltpu.VMEM((RNG, DH), jnp.float32),
        [pltpu.SemaphoreType.DMA for _ in range(2)],
        [pltpu.SemaphoreType.DMA for _ in range(2)],
    ],
)
def _sc_propagate(tab_hbm, src_hbm, dst_hbm, out_hbm,
                  se_b, de_b, cs_v, cl_v, rows_b, acc_v, gsem, ssem):
    c = lax.axis_index("c")
    s = lax.axis_index("s")
    lo = s * RNG

    zero16 = jnp.zeros((16,), dtype=jnp.float32)
    zero16i = jnp.zeros((16,), dtype=jnp.int32)

    @pl.loop(0, RNG)
    def _(i):
        for k in range(DH // 16):
            acc_v[i, pl.ds(k * 16, 16)] = zero16

    @pl.loop(0, C // 16)
    def _(i):
        cs_v[pl.ds(i * 16, 16)] = zero16i

    hc = tab_hbm.at[c]

    i16 = jnp.arange(16, dtype=jnp.int32)

    def flush(cnt):
        nb = lax.div(cnt + (GB - 1), GB)
        pltpu.async_copy(hc.at[cs_v.at[pl.ds(0, GB)]], rows_b[0], gsem[0])

        @pl.loop(0, lax.div(nb + 1, 2))
        def _(g):
            for j in range(2):
                b = g * 2 + j
                jn = 1 - j
                rows_v = rows_b[j]

                @pl.when(b < nb)
                def _():
                    base = b * GB
                    pltpu.make_async_copy(
                        hc.at[cs_v.at[pl.ds(base, GB)]], rows_v, gsem[j]
                    ).wait()

                    @pl.when(b + 1 < nb)
                    def _():
                        pltpu.async_copy(
                            hc.at[cs_v.at[pl.ds(base + GB, GB)]],
                            rows_b[jn], gsem[jn],
                        )

                    nin = jnp.minimum(cnt - base, GB)

                    @pl.loop(0, nin, step=16)
                    def _(i0):
                        ld16 = cl_v[pl.ds(base + i0, 16)]
                        nlane = nin - i0
                        for lane in range(16):
                            @pl.when(lane < nlane)
                            def _(lane=lane):
                                ld = ld16[lane]
                                for f in range(DH // 16):
                                    plsc.addupdate(
                                        acc_v.at[ld, pl.ds(16 * f, 16)],
                                        rows_v[i0 + lane, pl.ds(16 * f, 16)],
                                    )

    pltpu.async_copy(src_hbm.at[pl.ds(0, EB)], se_b[0], ssem[0])
    pltpu.async_copy(dst_hbm.at[pl.ds(0, EB)], de_b[0], ssem[0])

    @pl.loop(0, NEB // 2, init_carry=jnp.int32(0))
    def scan(g, cnt):
        for j in range(2):
            b = g * 2 + j
            jn = 1 - j
            pltpu.make_async_copy(src_hbm.at[pl.ds(0, EB)], se_b[j],
                                  ssem[j]).wait()
            pltpu.make_async_copy(dst_hbm.at[pl.ds(0, EB)], de_b[j],
                                  ssem[j]).wait()

            @pl.when(b + 1 < NEB)
            def _():
                pltpu.async_copy(src_hbm.at[pl.ds((b + 1) * EB, EB)],
                                 se_b[jn], ssem[jn])
                pltpu.async_copy(dst_hbm.at[pl.ds((b + 1) * EB, EB)],
                                 de_b[jn], ssem[jn])

            se_v, de_v = se_b[j], de_b[j]

            @pl.loop(0, EB // 16, init_carry=cnt)
            def vstep(v, cnt):
                full = cnt >= C - 16

                @pl.when(full)
                def _():
                    flush(cnt)

                cnt = jnp.where(full, 0, cnt)
                s16 = se_v[pl.ds(v * 16, 16)]
                d16 = de_v[pl.ds(v * 16, 16)]
                ld16 = d16 - lo
                m = (ld16 >= 0) & (ld16 < RNG)
                plsc.store_compressed(cs_v.at[pl.ds(cnt, 16)], s16, mask=m)
                plsc.store_compressed(cl_v.at[pl.ds(cnt, 16)], ld16, mask=m)
                return cnt + plsc.all_reduce_population_count(m)[0]

            cnt = vstep
        return cnt

    @pl.when(scan > 0)
    def _():
        flush(scan)

    pltpu.sync_copy(acc_v, out_hbm.at[c, pl.ds(s * RNG, RNG)])


_RB = 1024
_GRID = (N + _RB - 1) // _RB


def _dinv_from_deg(degp_blk):
    deg = jnp.sum(degp_blk, axis=0) + 1.0
    return lax.rsqrt(deg)


def _halves(ref):
    return jnp.concatenate([ref[0], ref[1]], axis=-1)


def _store_halves(o_ref, v):
    o_ref[0] = v[:, :DH]
    o_ref[1] = v[:, DH:]


def _tc_mm_scale_body(x_ref, w_ref, degp_ref, o_ref):
    dinv = _dinv_from_deg(degp_ref[...])
    h = jnp.dot(x_ref[...], w_ref[...], preferred_element_type=jnp.float32)
    _store_halves(o_ref, dinv[:, None] * h)


def _tc_mid_body(acc_ref, hp_ref, degp_ref, b1_ref, g_ref, be_ref, w2_ref,
                 o_ref):
    dinv = _dinv_from_deg(degp_ref[...])
    acc = _halves(acc_ref) + _halves(hp_ref)
    z = dinv[:, None] * acc + b1_ref[...]
    mu = jnp.mean(z, axis=-1, keepdims=True)
    zc = z - mu
    var = jnp.mean(zc * zc, axis=-1, keepdims=True)
    ln = zc * lax.rsqrt(var + 1e-5) * g_ref[...] + be_ref[...]
    g = jnp.maximum(ln, 0.0)
    h2 = jnp.dot(g, w2_ref[...], preferred_element_type=jnp.float32)
    _store_halves(o_ref, dinv[:, None] * h2)


def _tc_final_body(acc_ref, hp_ref, degp_ref, b2_ref, o_ref):
    dinv = _dinv_from_deg(degp_ref[...])
    acc = _halves(acc_ref) + _halves(hp_ref)
    o_ref[...] = jnp.maximum(dinv[:, None] * acc + b2_ref[...], 0.0)


_rowspec = pl.BlockSpec((_RB, D), lambda i: (i, 0))
_halfspec = pl.BlockSpec((NC, _RB, DH), lambda i: (0, i, 0))
_degspec = pl.BlockSpec((NW, _RB), lambda i: (0, i))
_wspec = pl.BlockSpec((D, H), lambda i: (0, 0))
_vspec = pl.BlockSpec((1, D), lambda i: (0, 0))
_out_half = jax.ShapeDtypeStruct((NC, N, DH), jnp.float32)

_tc_mm_scale = pl.pallas_call(
    _tc_mm_scale_body,
    grid=(_GRID,),
    in_specs=[_rowspec, _wspec, _degspec],
    out_specs=_halfspec,
    out_shape=_out_half,
)

_tc_mid = pl.pallas_call(
    _tc_mid_body,
    grid=(_GRID,),
    in_specs=[_halfspec, _halfspec, _degspec, _vspec, _vspec, _vspec, _wspec],
    out_specs=_halfspec,
    out_shape=_out_half,
)

_tc_final = pl.pallas_call(
    _tc_final_body,
    grid=(_GRID,),
    in_specs=[_halfspec, _halfspec, _degspec, _vspec],
    out_specs=_rowspec,
    out_shape=jax.ShapeDtypeStruct((N, D), jnp.float32),
)


def kernel(x, edge_index, W1, b1, gamma, beta, W2, b2):
    srcf = edge_index[0]
    dstf = edge_index[1]
    dst2 = dstf.reshape(NW, BPW, P)
    b1r = b1.reshape(1, H)
    gr = gamma.reshape(1, H)
    ber = beta.reshape(1, H)
    b2r = b2.reshape(1, D)

    degp = _sc_degree(dst2).reshape(NW, NP)
    h1p = _tc_mm_scale(x, W1, degp)
    acc1 = _sc_propagate(h1p, srcf, dstf)
    h2p = _tc_mid(acc1, h1p, degp, b1r, gr, ber, W2)
    acc2 = _sc_propagate(h2p, srcf, dstf)
    return _tc_final(acc2, h2p, degp, b2r)


# speedup vs baseline: 8.1530x; 1.4716x over previous
import functools

import jax
import jax.numpy as jnp
from jax import lax
from jax.experimental import pallas as pl
from jax.experimental.pallas import tpu as pltpu
from jax.experimental.pallas import tpu_sc as plsc

N = 10000
D = 128
H = 128
E = 320000

NC = 2
NS = 16
NW = NC * NS
DH = D // 2
NP = 10240
RNG = NP // NS
P = 80
EPW = E // NW
BPW = EPW // P
EB = 2000
NEB = E // EB
C = 4096
GB = 128

_mesh = plsc.VectorSubcoreMesh(core_axis_name="c", subcore_axis_name="s")


@functools.partial(
    pl.kernel,
    out_type=jax.ShapeDtypeStruct((NW, NP // 128, 128), jnp.float32),
    mesh=_mesh,
    compiler_params=pltpu.CompilerParams(needs_layout_passes=False),
    scratch_types=[
        pltpu.VMEM((BPW, P), jnp.int32),
        pltpu.VMEM((NP // 128, 128), jnp.float32),
    ],
)
def _sc_degree(dst2_hbm, deg_hbm, dst_v, hist_v):
    c = lax.axis_index("c")
    s = lax.axis_index("s")
    w = c * NS + s

    zero16 = jnp.zeros((16,), dtype=jnp.float32)
    one16 = jnp.full((16,), 1.0, dtype=jnp.float32)
    i16 = jnp.arange(16, dtype=jnp.int32)

    @pl.loop(0, NP // 128)
    def _(i):
        for k in range(8):
            hist_v[i, pl.ds(k * 16, 16)] = zero16

    pltpu.sync_copy(dst2_hbm.at[w], dst_v)

    @pl.loop(0, BPW)
    def _(r):
        for k in range(P // 16):
            d16 = dst_v[r, pl.ds(k * 16, 16)]
            row = lax.shift_right_logical(d16, 7)
            col = lax.bitwise_and(d16, 127)
            for lane in range(16):
                plsc.addupdate_scatter(
                    hist_v, [row, col], one16, mask=(i16 == lane)
                )

    pltpu.sync_copy(hist_v, deg_hbm.at[w])


@functools.partial(
    pl.kernel,
    out_type=jax.ShapeDtypeStruct((NC, NP, DH), jnp.float32),
    mesh=_mesh,
    compiler_params=pltpu.CompilerParams(
        use_tc_tiling_on_sc=False, needs_layout_passes=False
    ),
    scratch_types=[
        [pltpu.VMEM((EB,), jnp.int32) for _ in range(2)],
        [pltpu.VMEM((EB,), jnp.int32) for _ in range(2)],
        pltpu.VMEM((C,), jnp.int32),
        pltpu.VMEM((C,), jnp.int32),
        [pltpu.VMEM((GB, DH), jnp.float32) for _ in range(2)],
        pltpu.VMEM((RNG, DH), jnp.float32),
        [pltpu.SemaphoreType.DMA for _ in range(2)],
        [pltpu.SemaphoreType.DMA for _ in range(2)],
    ],
)
def _sc_propagate(tab_hbm, src_hbm, dst_hbm, out_hbm,
                  se_b, de_b, cs_v, cl_v, rows_b, acc_v, gsem, ssem):
    c = lax.axis_index("c")
    s = lax.axis_index("s")
    lo = s * RNG

    zero16 = jnp.zeros((16,), dtype=jnp.float32)
    zero16i = jnp.zeros((16,), dtype=jnp.int32)

    @pl.loop(0, RNG)
    def _(i):
        for k in range(DH // 16):
            acc_v[i, pl.ds(k * 16, 16)] = zero16

    @pl.loop(0, C // 16)
    def _(i):
        cs_v[pl.ds(i * 16, 16)] = zero16i

    hc = tab_hbm.at[c]

    i16 = jnp.arange(16, dtype=jnp.int32)

    def flush(cnt):
        nb = lax.div(cnt + (GB - 1), GB)
        pltpu.async_copy(hc.at[cs_v.at[pl.ds(0, GB)]], rows_b[0], gsem[0])

        @pl.loop(0, lax.div(nb + 1, 2))
        def _(g):
            for j in range(2):
                b = g * 2 + j
                jn = 1 - j
                rows_v = rows_b[j]

                @pl.when(b < nb)
                def _():
                    base = b * GB
                    pltpu.make_async_copy(
                        hc.at[cs_v.at[pl.ds(base, GB)]], rows_v, gsem[j]
                    ).wait()

                    @pl.when(b + 1 < nb)
                    def _():
                        pltpu.async_copy(
                            hc.at[cs_v.at[pl.ds(base + GB, GB)]],
                            rows_b[jn], gsem[jn],
                        )

                    nin = jnp.minimum(cnt - base, GB)
                    nfull = lax.bitwise_and(nin, -16)

                    @pl.loop(0, nfull, step=16)
                    def _(i0):
                        ld16 = cl_v[pl.ds(base + i0, 16)]
                        for lane in range(16):
                            ld = ld16[lane]
                            for f in range(DH // 16):
                                plsc.addupdate(
                                    acc_v.at[ld, pl.ds(16 * f, 16)],
                                    rows_v[i0 + lane, pl.ds(16 * f, 16)],
                                )

                    @pl.when(nfull < nin)
                    def _():
                        ld16 = cl_v[pl.ds(base + nfull, 16)]
                        nlane = nin - nfull
                        for lane in range(16):
                            @pl.when(lane < nlane)
                            def _(lane=lane):
                                ld = ld16[lane]
                                for f in range(DH // 16):
                                    plsc.addupdate(
                                        acc_v.at[ld, pl.ds(16 * f, 16)],
                                        rows_v[nfull + lane, pl.ds(16 * f, 16)],
                                    )

    pltpu.async_copy(src_hbm.at[pl.ds(0, EB)], se_b[0], ssem[0])
    pltpu.async_copy(dst_hbm.at[pl.ds(0, EB)], de_b[0], ssem[0])

    @pl.loop(0, NEB // 2, init_carry=jnp.int32(0))
    def scan(g, cnt):
        for j in range(2):
            b = g * 2 + j
            jn = 1 - j
            pltpu.make_async_copy(src_hbm.at[pl.ds(0, EB)], se_b[j],
                                  ssem[j]).wait()
            pltpu.make_async_copy(dst_hbm.at[pl.ds(0, EB)], de_b[j],
                                  ssem[j]).wait()

            @pl.when(b + 1 < NEB)
            def _():
                pltpu.async_copy(src_hbm.at[pl.ds((b + 1) * EB, EB)],
                                 se_b[jn], ssem[jn])
                pltpu.async_copy(dst_hbm.at[pl.ds((b + 1) * EB, EB)],
                                 de_b[jn], ssem[jn])

            se_v, de_v = se_b[j], de_b[j]

            full = cnt > C - EB

            @pl.when(full)
            def _():
                flush(cnt)

            cnt = jnp.where(full, 0, cnt)

            @pl.loop(0, EB // 16, init_carry=cnt, unroll=5)
            def vstep(v, cnt):
                s16 = se_v[pl.ds(v * 16, 16)]
                d16 = de_v[pl.ds(v * 16, 16)]
                ld16 = d16 - lo
                m = (ld16 >= 0) & (ld16 < RNG)
                plsc.store_compressed(cs_v.at[pl.ds(cnt, 16)], s16, mask=m)
                plsc.store_compressed(cl_v.at[pl.ds(cnt, 16)], ld16, mask=m)
                return cnt + plsc.all_reduce_population_count(m)[0]

            cnt = vstep
        return cnt

    @pl.when(scan > 0)
    def _():
        flush(scan)

    pltpu.sync_copy(acc_v, out_hbm.at[c, pl.ds(s * RNG, RNG)])


_RB = 1024
_GRID = (N + _RB - 1) // _RB


def _dinv_from_deg(degp_blk):
    deg = jnp.sum(degp_blk, axis=0) + 1.0
    return lax.rsqrt(deg)


def _halves(ref):
    return jnp.concatenate([ref[0], ref[1]], axis=-1)


def _store_halves(o_ref, v):
    o_ref[0] = v[:, :DH]
    o_ref[1] = v[:, DH:]


def _tc_mm_scale_body(x_ref, w_ref, degp_ref, o_ref):
    dinv = _dinv_from_deg(degp_ref[...])
    h = jnp.dot(x_ref[...], w_ref[...], preferred_element_type=jnp.float32)
    _store_halves(o_ref, dinv[:, None] * h)


def _tc_mid_body(acc_ref, hp_ref, degp_ref, b1_ref, g_ref, be_ref, w2_ref,
                 o_ref):
    dinv = _dinv_from_deg(degp_ref[...])
    acc = _halves(acc_ref) + _halves(hp_ref)
    z = dinv[:, None] * acc + b1_ref[...]
    mu = jnp.mean(z, axis=-1, keepdims=True)
    zc = z - mu
    var = jnp.mean(zc * zc, axis=-1, keepdims=True)
    ln = zc * lax.rsqrt(var + 1e-5) * g_ref[...] + be_ref[...]
    g = jnp.maximum(ln, 0.0)
    h2 = jnp.dot(g, w2_ref[...], preferred_element_type=jnp.float32)
    _store_halves(o_ref, dinv[:, None] * h2)


def _tc_final_body(acc_ref, hp_ref, degp_ref, b2_ref, o_ref):
    dinv = _dinv_from_deg(degp_ref[...])
    acc = _halves(acc_ref) + _halves(hp_ref)
    o_ref[...] = jnp.maximum(dinv[:, None] * acc + b2_ref[...], 0.0)


_rowspec = pl.BlockSpec((_RB, D), lambda i: (i, 0))
_halfspec = pl.BlockSpec((NC, _RB, DH), lambda i: (0, i, 0))
_degspec = pl.BlockSpec((NW, _RB), lambda i: (0, i))
_wspec = pl.BlockSpec((D, H), lambda i: (0, 0))
_vspec = pl.BlockSpec((1, D), lambda i: (0, 0))
_out_half = jax.ShapeDtypeStruct((NC, N, DH), jnp.float32)

_tc_mm_scale = pl.pallas_call(
    _tc_mm_scale_body,
    grid=(_GRID,),
    in_specs=[_rowspec, _wspec, _degspec],
    out_specs=_halfspec,
    out_shape=_out_half,
)

_tc_mid = pl.pallas_call(
    _tc_mid_body,
    grid=(_GRID,),
    in_specs=[_halfspec, _halfspec, _degspec, _vspec, _vspec, _vspec, _wspec],
    out_specs=_halfspec,
    out_shape=_out_half,
)

_tc_final = pl.pallas_call(
    _tc_final_body,
    grid=(_GRID,),
    in_specs=[_halfspec, _halfspec, _degspec, _vspec],
    out_specs=_rowspec,
    out_shape=jax.ShapeDtypeStruct((N, D), jnp.float32),
)


def kernel(x, edge_index, W1, b1, gamma, beta, W2, b2):
    srcf = edge_index[0]
    dstf = edge_index[1]
    dst2 = dstf.reshape(NW, BPW, P)
    b1r = b1.reshape(1, H)
    gr = gamma.reshape(1, H)
    ber = beta.reshape(1, H)
    b2r = b2.reshape(1, D)

    degp = _sc_degree(dst2).reshape(NW, NP)
    h1p = _tc_mm_scale(x, W1, degp)
    acc1 = _sc_propagate(h1p, srcf, dstf)
    h2p = _tc_mid(acc1, h1p, degp, b1r, gr, ber, W2)
    acc2 = _sc_propagate(h2p, srcf, dstf)
    return _tc_final(acc2, h2p, degp, b2r)


# speedup vs baseline: 9.6628x; 1.1852x over previous
import functools

import jax
import jax.numpy as jnp
from jax import lax
from jax.experimental import pallas as pl
from jax.experimental.pallas import tpu as pltpu
from jax.experimental.pallas import tpu_sc as plsc

N = 10000
D = 128
H = 128
E = 320000

NC = 2
NS = 16
NW = NC * NS
DH = D // 2
NP = 10240
RNG = NP // NS
P = 80
EPW = E // NW
BPW = EPW // P
EB = 2000
NEB = E // EB
C = 4096
GB = 128

_mesh = plsc.VectorSubcoreMesh(core_axis_name="c", subcore_axis_name="s")


@functools.partial(
    pl.kernel,
    out_type=jax.ShapeDtypeStruct((NW, NP // 128, 128), jnp.float32),
    mesh=_mesh,
    compiler_params=pltpu.CompilerParams(needs_layout_passes=False),
    scratch_types=[
        pltpu.VMEM((BPW, P), jnp.int32),
        pltpu.VMEM((NP // 128, 128), jnp.float32),
    ],
)
def _sc_degree(dst2_hbm, deg_hbm, dst_v, hist_v):
    c = lax.axis_index("c")
    s = lax.axis_index("s")
    w = c * NS + s

    zero16 = jnp.zeros((16,), dtype=jnp.float32)
    one16 = jnp.full((16,), 1.0, dtype=jnp.float32)
    i16 = jnp.arange(16, dtype=jnp.int32)

    @pl.loop(0, NP // 128)
    def _(i):
        for k in range(8):
            hist_v[i, pl.ds(k * 16, 16)] = zero16

    pltpu.sync_copy(dst2_hbm.at[w], dst_v)

    @pl.loop(0, BPW)
    def _(r):
        for k in range(P // 16):
            d16 = dst_v[r, pl.ds(k * 16, 16)]
            row = lax.shift_right_logical(d16, 7)
            col = lax.bitwise_and(d16, 127)
            for lane in range(16):
                plsc.addupdate_scatter(
                    hist_v, [row, col], one16, mask=(i16 == lane)
                )

    pltpu.sync_copy(hist_v, deg_hbm.at[w])


MAXF = 80
NEB2 = NEB // 2

_sc_params = pltpu.CompilerParams(
    use_tc_tiling_on_sc=False, needs_layout_passes=False
)


@functools.partial(
    pl.kernel,
    out_type=(
        jax.ShapeDtypeStruct((NW, MAXF, C), jnp.int32),
        jax.ShapeDtypeStruct((NW, 1, 128), jnp.int32),
    ),
    mesh=_mesh,
    compiler_params=_sc_params,
    scratch_types=[
        [pltpu.VMEM((EB,), jnp.int32) for _ in range(2)],
        [pltpu.VMEM((EB,), jnp.int32) for _ in range(2)],
        pltpu.VMEM((C,), jnp.int32),
        pltpu.VMEM((1, 128), jnp.int32),
        [pltpu.SemaphoreType.DMA for _ in range(2)],
    ],
)
def _sc_prep(src_hbm, dst_hbm, lists_hbm, counts_hbm,
             se_b, de_b, pk_v, cv_v, ssem):
    c = lax.axis_index("c")
    s = lax.axis_index("s")
    w = s * 2 + c
    lo = s * RNG
    base_blk = c * NEB2

    zero16i = jnp.zeros((16,), dtype=jnp.int32)
    i16 = jnp.arange(16, dtype=jnp.int32)

    for k in range(128 // 16):
        cv_v[0, pl.ds(k * 16, 16)] = zero16i

    @pl.loop(0, C // 16)
    def _(i):
        pk_v[pl.ds(i * 16, 16)] = zero16i

    def writeout(cnt, k):
        pltpu.sync_copy(pk_v, lists_hbm.at[w, k])
        kb = lax.bitwise_and(k, -16)
        v16 = cv_v[0, pl.ds(kb, 16)]
        cv_v[0, pl.ds(kb, 16)] = jnp.where(i16 == k - kb, i16 * 0 + cnt, v16)

    pltpu.async_copy(src_hbm.at[pl.ds(base_blk * EB, EB)], se_b[0], ssem[0])
    pltpu.async_copy(dst_hbm.at[pl.ds(base_blk * EB, EB)], de_b[0], ssem[0])

    @pl.loop(0, NEB2 // 2, init_carry=(jnp.int32(0), jnp.int32(0)))
    def scan(g, carry):
        cnt, k = carry
        for j in range(2):
            b = g * 2 + j
            jn = 1 - j
            pltpu.make_async_copy(src_hbm.at[pl.ds(0, EB)], se_b[j],
                                  ssem[j]).wait()
            pltpu.make_async_copy(dst_hbm.at[pl.ds(0, EB)], de_b[j],
                                  ssem[j]).wait()

            @pl.when(b + 1 < NEB2)
            def _():
                nxt = (base_blk + b + 1) * EB
                pltpu.async_copy(src_hbm.at[pl.ds(nxt, EB)], se_b[jn],
                                 ssem[jn])
                pltpu.async_copy(dst_hbm.at[pl.ds(nxt, EB)], de_b[jn],
                                 ssem[jn])

            se_v, de_v = se_b[j], de_b[j]

            full = cnt > C - EB

            @pl.when(full)
            def _():
                writeout(cnt, k)

            k = jnp.where(full, k + 1, k)
            cnt = jnp.where(full, 0, cnt)

            @pl.loop(0, EB // 16, init_carry=cnt, unroll=5)
            def vstep(v, cnt):
                s16 = se_v[pl.ds(v * 16, 16)]
                d16 = de_v[pl.ds(v * 16, 16)]
                ld16 = d16 - lo
                m = (ld16 >= 0) & (ld16 < RNG)
                pk16 = s16 + lax.shift_left(ld16, 14)
                plsc.store_compressed(pk_v.at[pl.ds(cnt, 16)], pk16, mask=m)
                return cnt + plsc.all_reduce_population_count(m)[0]

            cnt = vstep
        return (cnt, k)

    cnt, k = scan

    @pl.when(cnt > 0)
    def _():
        writeout(cnt, k)

    pltpu.sync_copy(cv_v, counts_hbm.at[w])


@functools.partial(
    pl.kernel,
    out_type=jax.ShapeDtypeStruct((NC, NP, DH), jnp.float32),
    mesh=_mesh,
    compiler_params=_sc_params,
    scratch_types=[
        pltpu.VMEM((C,), jnp.int32),
        pltpu.VMEM((C,), jnp.int32),
        pltpu.VMEM((1, 128), jnp.int32),
        [pltpu.VMEM((GB, DH), jnp.float32) for _ in range(2)],
        pltpu.VMEM((RNG, DH), jnp.float32),
        [pltpu.SemaphoreType.DMA for _ in range(2)],
    ],
)
def _sc_propagate(tab_hbm, lists_hbm, counts_hbm, out_hbm,
                  pk_v, cs_v, cv_v, rows_b, acc_v, gsem):
    c = lax.axis_index("c")
    s = lax.axis_index("s")

    zero16 = jnp.zeros((16,), dtype=jnp.float32)
    i16 = jnp.arange(16, dtype=jnp.int32)

    @pl.loop(0, RNG)
    def _(i):
        for k in range(DH // 16):
            acc_v[i, pl.ds(k * 16, 16)] = zero16

    hc = tab_hbm.at[c]

    def accumulate(cnt):
        nb = lax.div(cnt + (GB - 1), GB)
        pltpu.async_copy(hc.at[cs_v.at[pl.ds(0, GB)]], rows_b[0], gsem[0])

        @pl.loop(0, lax.div(nb + 1, 2))
        def _(g):
            for j in range(2):
                b = g * 2 + j
                jn = 1 - j
                rows_v = rows_b[j]

                @pl.when(b < nb)
                def _():
                    base = b * GB
                    pltpu.make_async_copy(
                        hc.at[cs_v.at[pl.ds(base, GB)]], rows_v, gsem[j]
                    ).wait()

                    @pl.when(b + 1 < nb)
                    def _():
                        pltpu.async_copy(
                            hc.at[cs_v.at[pl.ds(base + GB, GB)]],
                            rows_b[jn], gsem[jn],
                        )

                    nin = jnp.minimum(cnt - base, GB)
                    nfull = lax.bitwise_and(nin, -16)

                    @pl.loop(0, nfull, step=16)
                    def _(i0):
                        ld16 = lax.shift_right_logical(
                            pk_v[pl.ds(base + i0, 16)], 14)
                        for lane in range(16):
                            ld = ld16[lane]
                            for f in range(DH // 16):
                                plsc.addupdate(
                                    acc_v.at[ld, pl.ds(16 * f, 16)],
                                    rows_v[i0 + lane, pl.ds(16 * f, 16)],
                                )

                    @pl.when(nfull < nin)
                    def _():
                        ld16 = lax.shift_right_logical(
                            pk_v[pl.ds(base + nfull, 16)], 14)
                        nlane = nin - nfull
                        for lane in range(16):
                            @pl.when(lane < nlane)
                            def _(lane=lane):
                                ld = ld16[lane]
                                for f in range(DH // 16):
                                    plsc.addupdate(
                                        acc_v.at[ld, pl.ds(16 * f, 16)],
                                        rows_v[nfull + lane, pl.ds(16 * f, 16)],
                                    )

    for h in range(2):
        wl = s * 2 + h
        pltpu.sync_copy(counts_hbm.at[wl], cv_v)

        @pl.loop(0, MAXF)
        def _(k):
            kb = lax.bitwise_and(k, -16)
            v16 = cv_v[0, pl.ds(kb, 16)]
            cntk = jnp.sum(jnp.where(i16 == k - kb, v16, 0))

            @pl.when(cntk > 0)
            def _():
                pltpu.sync_copy(lists_hbm.at[wl, k], pk_v)

                @pl.loop(0, C // 16)
                def _(i):
                    cs_v[pl.ds(i * 16, 16)] = lax.bitwise_and(
                        pk_v[pl.ds(i * 16, 16)], 16383)

                accumulate(cntk)

    pltpu.sync_copy(acc_v, out_hbm.at[c, pl.ds(s * RNG, RNG)])


_RB = 1024
_GRID = (N + _RB - 1) // _RB


def _dinv_from_deg(degp_blk):
    deg = jnp.sum(degp_blk, axis=0) + 1.0
    return lax.rsqrt(deg)


def _halves(ref):
    return jnp.concatenate([ref[0], ref[1]], axis=-1)


def _store_halves(o_ref, v):
    o_ref[0] = v[:, :DH]
    o_ref[1] = v[:, DH:]


def _tc_mm_scale_body(x_ref, w_ref, degp_ref, o_ref):
    dinv = _dinv_from_deg(degp_ref[...])
    h = jnp.dot(x_ref[...], w_ref[...], preferred_element_type=jnp.float32)
    _store_halves(o_ref, dinv[:, None] * h)


def _tc_mid_body(acc_ref, hp_ref, degp_ref, b1_ref, g_ref, be_ref, w2_ref,
                 o_ref):
    dinv = _dinv_from_deg(degp_ref[...])
    acc = _halves(acc_ref) + _halves(hp_ref)
    z = dinv[:, None] * acc + b1_ref[...]
    mu = jnp.mean(z, axis=-1, keepdims=True)
    zc = z - mu
    var = jnp.mean(zc * zc, axis=-1, keepdims=True)
    ln = zc * lax.rsqrt(var + 1e-5) * g_ref[...] + be_ref[...]
    g = jnp.maximum(ln, 0.0)
    h2 = jnp.dot(g, w2_ref[...], preferred_element_type=jnp.float32)
    _store_halves(o_ref, dinv[:, None] * h2)


def _tc_final_body(acc_ref, hp_ref, degp_ref, b2_ref, o_ref):
    dinv = _dinv_from_deg(degp_ref[...])
    acc = _halves(acc_ref) + _halves(hp_ref)
    o_ref[...] = jnp.maximum(dinv[:, None] * acc + b2_ref[...], 0.0)


_rowspec = pl.BlockSpec((_RB, D), lambda i: (i, 0))
_halfspec = pl.BlockSpec((NC, _RB, DH), lambda i: (0, i, 0))
_degspec = pl.BlockSpec((NW, _RB), lambda i: (0, i))
_wspec = pl.BlockSpec((D, H), lambda i: (0, 0))
_vspec = pl.BlockSpec((1, D), lambda i: (0, 0))
_out_half = jax.ShapeDtypeStruct((NC, N, DH), jnp.float32)

_tc_mm_scale = pl.pallas_call(
    _tc_mm_scale_body,
    grid=(_GRID,),
    in_specs=[_rowspec, _wspec, _degspec],
    out_specs=_halfspec,
    out_shape=_out_half,
)

_tc_mid = pl.pallas_call(
    _tc_mid_body,
    grid=(_GRID,),
    in_specs=[_halfspec, _halfspec, _degspec, _vspec, _vspec, _vspec, _wspec],
    out_specs=_halfspec,
    out_shape=_out_half,
)

_tc_final = pl.pallas_call(
    _tc_final_body,
    grid=(_GRID,),
    in_specs=[_halfspec, _halfspec, _degspec, _vspec],
    out_specs=_rowspec,
    out_shape=jax.ShapeDtypeStruct((N, D), jnp.float32),
)


def kernel(x, edge_index, W1, b1, gamma, beta, W2, b2):
    srcf = edge_index[0]
    dstf = edge_index[1]
    dst2 = dstf.reshape(NW, BPW, P)
    b1r = b1.reshape(1, H)
    gr = gamma.reshape(1, H)
    ber = beta.reshape(1, H)
    b2r = b2.reshape(1, D)

    lists, counts = _sc_prep(srcf, dstf)
    degp = _sc_degree(dst2).reshape(NW, NP)
    h1p = _tc_mm_scale(x, W1, degp)
    acc1 = _sc_propagate(h1p, lists, counts)
    h2p = _tc_mid(acc1, h1p, degp, b1r, gr, ber, W2)
    acc2 = _sc_propagate(h2p, lists, counts)
    return _tc_final(acc2, h2p, degp, b2r)
